# Initial kernel scaffold; baseline (speedup 1.0000x reference)
import jax
import jax.numpy as jnp
from jax.experimental import pallas as pl


def kernel(x, edge_index, graph_id, W1, b1, W2, b2, Wr1, br1, Wr2, br2, Wo, bo):
    raise NotImplementedError("write your pallas kernel here")


# speedup vs baseline: 15.1433x; 15.1433x over previous
import functools

import jax
import jax.numpy as jnp
from jax import lax
from jax.experimental import pallas as pl
from jax.experimental.pallas import tpu as pltpu
from jax.experimental.pallas import tpu_sc as plsc

N = 10000
E = 320000
D = 128
H = 256
HH = 128

NPAD = 10240

DEG_CHUNK = 80
DEG_ROWS_PER_SUB = 125

AGG_CHUNK = 125
AGG_ROWS_PER_SUB = 160
ROWS_PER_TILE = 640

_MESH = plsc.VectorSubcoreMesh(core_axis_name="c", subcore_axis_name="s")


@functools.partial(
    pl.kernel,
    out_type=jax.ShapeDtypeStruct((2, NPAD), jnp.float32),
    mesh=_MESH,
    scratch_types=[
        pltpu.VMEM((DEG_ROWS_PER_SUB, DEG_CHUNK), jnp.int32),
        pltpu.VMEM((DEG_CHUNK,), jnp.float32),
        pltpu.VMEM((640,), jnp.float32),
        pltpu.VMEM_SHARED((NPAD,), jnp.float32),
    ],
)
def _sc_deg(dst_hbm, out_hbm, idx_v, ones_v, zero_v, deg_sh):
    c = lax.axis_index("c")
    s = lax.axis_index("s")

    pltpu.sync_copy(dst_hbm.at[c, s], idx_v)

    def _fill_ones(k, _):
        ones_v[pl.ds(pl.multiple_of(k * 16, 16), 16)] = jnp.ones((16,), jnp.float32)
        return 0
    lax.fori_loop(0, DEG_CHUNK // 16, _fill_ones, 0)

    def _fill_zero(k, _):
        zero_v[pl.ds(pl.multiple_of(k * 16, 16), 16)] = jnp.zeros((16,), jnp.float32)
        return 0
    lax.fori_loop(0, 40, _fill_zero, 0)

    pltpu.sync_copy(zero_v, deg_sh.at[pl.ds(pl.multiple_of(s * 640, 8), 640)])
    plsc.subcore_barrier()

    def _scat(j, _):
        pltpu.sync_copy(ones_v, deg_sh.at[idx_v.at[j]], add=True)
        return 0
    lax.fori_loop(0, DEG_ROWS_PER_SUB, _scat, 0)

    plsc.subcore_barrier()

    @pl.when(s == 0)
    def _():
        pltpu.sync_copy(deg_sh, out_hbm.at[c])


@functools.partial(
    pl.kernel,
    out_type=jax.ShapeDtypeStruct((2, NPAD, HH), jnp.float32),
    mesh=_MESH,
    scratch_types=[
        pltpu.VMEM((16, AGG_CHUNK), jnp.int32),
        pltpu.VMEM((16, AGG_CHUNK), jnp.int32),
        pltpu.VMEM((AGG_CHUNK, HH), jnp.float32),
        pltpu.VMEM_SHARED((NPAD, HH), jnp.float32),
        pltpu.SemaphoreType.DMA,
    ],
)
def _sc_agg(p_hbm, srcidx_hbm, dstidx_hbm, zeros_hbm, out_hbm,
            src_v, dst_v, rows_v, acc_sh, sem):
    c = lax.axis_index("c")
    s = lax.axis_index("s")

    base = s * AGG_ROWS_PER_SUB

    r0 = pl.multiple_of(s * ROWS_PER_TILE, 8)
    pltpu.sync_copy(zeros_hbm.at[pl.ds(r0, ROWS_PER_TILE)],
                    acc_sh.at[pl.ds(r0, ROWS_PER_TILE)])
    plsc.subcore_barrier()

    def _group(g, _):
        off = pl.multiple_of(base + g * 16, 8)
        pltpu.sync_copy(srcidx_hbm.at[c, pl.ds(off, 16)], src_v)
        pltpu.sync_copy(dstidx_hbm.at[pl.ds(off, 16)], dst_v)

        def _edge(j, _):
            pltpu.async_copy(p_hbm.at[src_v.at[j]], rows_v, sem).wait()
            pltpu.sync_copy(rows_v, acc_sh.at[dst_v.at[j]], add=True)
            return 0
        lax.fori_loop(0, 16, _edge, 0)
        return 0
    lax.fori_loop(0, AGG_ROWS_PER_SUB // 16, _group, 0)

    plsc.subcore_barrier()
    pltpu.sync_copy(acc_sh.at[pl.ds(r0, ROWS_PER_TILE)],
                    out_hbm.at[c, pl.ds(r0, ROWS_PER_TILE)])


_BLK = 1000


def _mm1_body(x_ref, w_ref, b_ref, deg_ref, p_ref):
    dinv = lax.rsqrt(deg_ref[:, 0:1] + deg_ref[:, 1:2] + 1.0)
    h = jnp.dot(x_ref[...], w_ref[...], preferred_element_type=jnp.float32)
    p_ref[0] = (h + b_ref[0, 0:1, :]) * dinv


def _tc_mm1(x, W1, b1r, degT):
    return pl.pallas_call(
        _mm1_body,
        grid=(2, N // _BLK),
        in_specs=[
            pl.BlockSpec((_BLK, D), lambda c, i: (i, 0)),
            pl.BlockSpec((D, HH), lambda c, i: (0, c)),
            pl.BlockSpec((1, 8, HH), lambda c, i: (c, 0, 0)),
            pl.BlockSpec((_BLK, 2), lambda c, i: (i, 0)),
        ],
        out_specs=pl.BlockSpec((1, _BLK, HH), lambda c, i: (c, i, 0)),
        out_shape=jax.ShapeDtypeStruct((2, N, HH), jnp.float32),
    )(x, W1, b1r, degT)


def _mm2_body(acc_ref, p_ref, deg_ref, w_ref, b_ref, out_ref):
    dinv = lax.rsqrt(deg_ref[:, 0:1] + deg_ref[:, 1:2] + 1.0)
    h0 = jnp.maximum(dinv * (acc_ref[0] + p_ref[0]), 0.0)
    h1 = jnp.maximum(dinv * (acc_ref[1] + p_ref[1]), 0.0)
    h = jnp.concatenate([h0, h1], axis=1)
    out_ref[0] = (jnp.dot(h, w_ref[...], preferred_element_type=jnp.float32)
                  + b_ref[0, 0:1, :]) * dinv


def _tc_mm2(acc1, p1, degT, W2, b2r):
    return pl.pallas_call(
        _mm2_body,
        grid=(2, N // _BLK),
        in_specs=[
            pl.BlockSpec((2, _BLK, HH), lambda c, i: (0, i, 0)),
            pl.BlockSpec((2, _BLK, HH), lambda c, i: (0, i, 0)),
            pl.BlockSpec((_BLK, 2), lambda c, i: (i, 0)),
            pl.BlockSpec((H, HH), lambda c, i: (0, c)),
            pl.BlockSpec((1, 8, HH), lambda c, i: (c, 0, 0)),
        ],
        out_specs=pl.BlockSpec((1, _BLK, HH), lambda c, i: (c, i, 0)),
        out_shape=jax.ShapeDtypeStruct((2, N, HH), jnp.float32),
    )(acc1, p1, degT, W2, b2r)


def _final_body(acc_ref, p_ref, deg_ref, wr1_ref, br1_ref, wr2_ref, br2_ref,
                wo_ref, bo_ref, out_ref, sum_ref):
    i = pl.program_id(0)
    dinv = lax.rsqrt(deg_ref[:, 0:1] + deg_ref[:, 1:2] + 1.0)
    h0 = jnp.maximum(dinv * (acc_ref[0] + p_ref[0]), 0.0)
    h1 = jnp.maximum(dinv * (acc_ref[1] + p_ref[1]), 0.0)
    part = jnp.concatenate(
        [jnp.sum(h0, axis=0, keepdims=True), jnp.sum(h1, axis=0, keepdims=True)],
        axis=1)

    @pl.when(i == 0)
    def _():
        sum_ref[0:1] = part

    @pl.when(i > 0)
    def _():
        sum_ref[0:1] = sum_ref[0:1] + part

    @pl.when(i == pl.num_programs(0) - 1)
    def _():
        pooled = sum_ref[0:1] * (1.0 / N)
        r = jnp.dot(pooled, wr1_ref[...],
                    preferred_element_type=jnp.float32) + br1_ref[...]
        r = jnp.dot(r, wr2_ref[...],
                    preferred_element_type=jnp.float32) + br2_ref[...]
        out_ref[...] = jnp.dot(r, wo_ref[...],
                               preferred_element_type=jnp.float32) + bo_ref[...]


def _tc_final(acc2, p2, degT, Wr1, br1r, Wr2, br2r, Wo, bor):
    return pl.pallas_call(
        _final_body,
        grid=(N // _BLK,),
        in_specs=[
            pl.BlockSpec((2, _BLK, HH), lambda i: (0, i, 0)),
            pl.BlockSpec((2, _BLK, HH), lambda i: (0, i, 0)),
            pl.BlockSpec((_BLK, 2), lambda i: (i, 0)),
            pl.BlockSpec((H, H), lambda i: (0, 0)),
            pl.BlockSpec((1, H), lambda i: (0, 0)),
            pl.BlockSpec((H, H), lambda i: (0, 0)),
            pl.BlockSpec((1, H), lambda i: (0, 0)),
            pl.BlockSpec((H, 1), lambda i: (0, 0)),
            pl.BlockSpec((1, 1), lambda i: (0, 0)),
        ],
        out_specs=pl.BlockSpec((1, 1), lambda i: (0, 0)),
        out_shape=jax.ShapeDtypeStruct((1, 1), jnp.float32),
        scratch_shapes=[pltpu.VMEM((8, H), jnp.float32)],
    )(acc2, p2, degT, Wr1, br1r, Wr2, br2r, Wo, bor)


def kernel(x, edge_index, graph_id, W1, b1, W2, b2, Wr1, br1, Wr2, br2, Wo, bo):
    del graph_id
    src = edge_index[0]
    dst = edge_index[1]

    dst_deg = dst.reshape(2, 16, DEG_ROWS_PER_SUB, DEG_CHUNK)
    src2d = src.reshape(E // AGG_CHUNK, AGG_CHUNK)
    src_idx2 = jnp.stack([src2d, src2d + N])
    dst2d = dst.reshape(E // AGG_CHUNK, AGG_CHUNK)
    zeros_b = jnp.zeros((NPAD, HH), jnp.float32)

    deg2 = _sc_deg(dst_deg)
    degT = deg2[:, :N].T

    b1r = jnp.broadcast_to(b1.reshape(2, 1, HH), (2, 8, HH))
    b2r = jnp.broadcast_to(b2.reshape(2, 1, HH), (2, 8, HH))

    p1 = _tc_mm1(x, W1, b1r, degT)
    acc1 = _sc_agg(p1.reshape(2 * N, HH), src_idx2, dst2d, zeros_b)
    p2 = _tc_mm2(acc1, p1, degT, W2, b2r)
    acc2 = _sc_agg(p2.reshape(2 * N, HH), src_idx2, dst2d, zeros_b)
    return _tc_final(acc2, p2, degT, Wr1, br1.reshape(1, H), Wr2,
                     br2.reshape(1, H), Wo, bo.reshape(1, 1))


# speedup vs baseline: 22.0013x; 1.4529x over previous
import functools

import jax
import jax.numpy as jnp
from jax import lax
from jax.experimental import pallas as pl
from jax.experimental.pallas import tpu as pltpu
from jax.experimental.pallas import tpu_sc as plsc

N = 10000
E = 320000
D = 128
H = 256
HH = 128

NPAD = 10240

DEG_CHUNK = 80
DEG_ROWS_PER_SUB = 125

AGG_CHUNK = 125
AGG_ROWS_PER_SUB = 160
ROWS_PER_TILE = 640

_MESH = plsc.VectorSubcoreMesh(core_axis_name="c", subcore_axis_name="s")


@functools.partial(
    pl.kernel,
    out_type=jax.ShapeDtypeStruct((2, NPAD), jnp.float32),
    mesh=_MESH,
    scratch_types=[
        pltpu.VMEM((DEG_ROWS_PER_SUB, DEG_CHUNK), jnp.int32),
        pltpu.VMEM((DEG_CHUNK,), jnp.float32),
        pltpu.VMEM((640,), jnp.float32),
        pltpu.VMEM_SHARED((NPAD,), jnp.float32),
    ],
)
def _sc_deg(dst_hbm, out_hbm, idx_v, ones_v, zero_v, deg_sh):
    c = lax.axis_index("c")
    s = lax.axis_index("s")

    pltpu.sync_copy(dst_hbm.at[c, s], idx_v)

    def _fill_ones(k, _):
        ones_v[pl.ds(pl.multiple_of(k * 16, 16), 16)] = jnp.ones((16,), jnp.float32)
        return 0
    lax.fori_loop(0, DEG_CHUNK // 16, _fill_ones, 0)

    def _fill_zero(k, _):
        zero_v[pl.ds(pl.multiple_of(k * 16, 16), 16)] = jnp.zeros((16,), jnp.float32)
        return 0
    lax.fori_loop(0, 40, _fill_zero, 0)

    pltpu.sync_copy(zero_v, deg_sh.at[pl.ds(pl.multiple_of(s * 640, 8), 640)])
    plsc.subcore_barrier()

    def _scat(j, _):
        pltpu.sync_copy(ones_v, deg_sh.at[idx_v.at[j]], add=True)
        return 0
    lax.fori_loop(0, DEG_ROWS_PER_SUB, _scat, 0)

    plsc.subcore_barrier()

    @pl.when(s == 0)
    def _():
        pltpu.sync_copy(deg_sh, out_hbm.at[c])


@functools.partial(
    pl.kernel,
    out_type=jax.ShapeDtypeStruct((2, NPAD, HH), jnp.float32),
    mesh=_MESH,
    scratch_types=[
        pltpu.VMEM((16, AGG_CHUNK), jnp.int32),
        pltpu.VMEM((16, AGG_CHUNK), jnp.int32),
        pltpu.VMEM((AGG_CHUNK, HH), jnp.float32),
        pltpu.VMEM((AGG_CHUNK, HH), jnp.float32),
        pltpu.VMEM_SHARED((NPAD, HH), jnp.float32),
        pltpu.SemaphoreType.DMA,
        pltpu.SemaphoreType.DMA,
    ],
)
def _sc_agg(p_hbm, srcidx_hbm, dstidx_hbm, zeros_hbm, out_hbm,
            src_v, dst_v, rows0_v, rows1_v, acc_sh, sem0, sem1):
    c = lax.axis_index("c")
    s = lax.axis_index("s")

    base = s * AGG_ROWS_PER_SUB

    r0 = pl.multiple_of(s * ROWS_PER_TILE, 8)
    pltpu.sync_copy(zeros_hbm.at[pl.ds(r0, ROWS_PER_TILE)],
                    acc_sh.at[pl.ds(r0, ROWS_PER_TILE)])
    plsc.subcore_barrier()

    rows = (rows0_v, rows1_v)
    sems = (sem0, sem1)

    def _group(g, _):
        off = pl.multiple_of(base + g * 16, 8)
        pltpu.sync_copy(srcidx_hbm.at[c, pl.ds(off, 16)], src_v)
        pltpu.sync_copy(dstidx_hbm.at[pl.ds(off, 16)], dst_v)

        gd = [None, None]
        gd[0] = pltpu.async_copy(p_hbm.at[src_v.at[0]], rows[0], sems[0])
        for j in range(16):
            b = j & 1
            if j < 15:
                gd[1 - b] = pltpu.async_copy(
                    p_hbm.at[src_v.at[j + 1]], rows[1 - b], sems[1 - b])
            gd[b].wait()
            pltpu.sync_copy(rows[b], acc_sh.at[dst_v.at[j]], add=True)
        return 0
    lax.fori_loop(0, AGG_ROWS_PER_SUB // 16, _group, 0)

    plsc.subcore_barrier()
    pltpu.sync_copy(acc_sh.at[pl.ds(r0, ROWS_PER_TILE)],
                    out_hbm.at[c, pl.ds(r0, ROWS_PER_TILE)])


_BLK = 1000


def _mm1_body(x_ref, w_ref, b_ref, deg_ref, p_ref):
    dinv = lax.rsqrt(deg_ref[:, 0:1] + deg_ref[:, 1:2] + 1.0)
    h = jnp.dot(x_ref[...], w_ref[...], preferred_element_type=jnp.float32)
    p_ref[0] = (h + b_ref[0, 0:1, :]) * dinv


def _tc_mm1(x, W1, b1r, degT):
    return pl.pallas_call(
        _mm1_body,
        grid=(2, N // _BLK),
        in_specs=[
            pl.BlockSpec((_BLK, D), lambda c, i: (i, 0)),
            pl.BlockSpec((D, HH), lambda c, i: (0, c)),
            pl.BlockSpec((1, 8, HH), lambda c, i: (c, 0, 0)),
            pl.BlockSpec((_BLK, 2), lambda c, i: (i, 0)),
        ],
        out_specs=pl.BlockSpec((1, _BLK, HH), lambda c, i: (c, i, 0)),
        out_shape=jax.ShapeDtypeStruct((2, N, HH), jnp.float32),
    )(x, W1, b1r, degT)


def _mm2_body(acc_ref, p_ref, deg_ref, w_ref, b_ref, out_ref):
    dinv = lax.rsqrt(deg_ref[:, 0:1] + deg_ref[:, 1:2] + 1.0)
    h0 = jnp.maximum(dinv * (acc_ref[0] + p_ref[0]), 0.0)
    h1 = jnp.maximum(dinv * (acc_ref[1] + p_ref[1]), 0.0)
    h = jnp.concatenate([h0, h1], axis=1)
    out_ref[0] = (jnp.dot(h, w_ref[...], preferred_element_type=jnp.float32)
                  + b_ref[0, 0:1, :]) * dinv


def _tc_mm2(acc1, p1, degT, W2, b2r):
    return pl.pallas_call(
        _mm2_body,
        grid=(2, N // _BLK),
        in_specs=[
            pl.BlockSpec((2, _BLK, HH), lambda c, i: (0, i, 0)),
            pl.BlockSpec((2, _BLK, HH), lambda c, i: (0, i, 0)),
            pl.BlockSpec((_BLK, 2), lambda c, i: (i, 0)),
            pl.BlockSpec((H, HH), lambda c, i: (0, c)),
            pl.BlockSpec((1, 8, HH), lambda c, i: (c, 0, 0)),
        ],
        out_specs=pl.BlockSpec((1, _BLK, HH), lambda c, i: (c, i, 0)),
        out_shape=jax.ShapeDtypeStruct((2, N, HH), jnp.float32),
    )(acc1, p1, degT, W2, b2r)


def _final_body(acc_ref, p_ref, deg_ref, wr1_ref, br1_ref, wr2_ref, br2_ref,
                wo_ref, bo_ref, out_ref, sum_ref):
    i = pl.program_id(0)
    dinv = lax.rsqrt(deg_ref[:, 0:1] + deg_ref[:, 1:2] + 1.0)
    h0 = jnp.maximum(dinv * (acc_ref[0] + p_ref[0]), 0.0)
    h1 = jnp.maximum(dinv * (acc_ref[1] + p_ref[1]), 0.0)
    part = jnp.concatenate(
        [jnp.sum(h0, axis=0, keepdims=True), jnp.sum(h1, axis=0, keepdims=True)],
        axis=1)

    @pl.when(i == 0)
    def _():
        sum_ref[0:1] = part

    @pl.when(i > 0)
    def _():
        sum_ref[0:1] = sum_ref[0:1] + part

    @pl.when(i == pl.num_programs(0) - 1)
    def _():
        pooled = sum_ref[0:1] * (1.0 / N)
        r = jnp.dot(pooled, wr1_ref[...],
                    preferred_element_type=jnp.float32) + br1_ref[...]
        r = jnp.dot(r, wr2_ref[...],
                    preferred_element_type=jnp.float32) + br2_ref[...]
        out_ref[...] = jnp.dot(r, wo_ref[...],
                               preferred_element_type=jnp.float32) + bo_ref[...]


def _tc_final(acc2, p2, degT, Wr1, br1r, Wr2, br2r, Wo, bor):
    return pl.pallas_call(
        _final_body,
        grid=(N // _BLK,),
        in_specs=[
            pl.BlockSpec((2, _BLK, HH), lambda i: (0, i, 0)),
            pl.BlockSpec((2, _BLK, HH), lambda i: (0, i, 0)),
            pl.BlockSpec((_BLK, 2), lambda i: (i, 0)),
            pl.BlockSpec((H, H), lambda i: (0, 0)),
            pl.BlockSpec((1, H), lambda i: (0, 0)),
            pl.BlockSpec((H, H), lambda i: (0, 0)),
            pl.BlockSpec((1, H), lambda i: (0, 0)),
            pl.BlockSpec((H, 1), lambda i: (0, 0)),
            pl.BlockSpec((1, 1), lambda i: (0, 0)),
        ],
        out_specs=pl.BlockSpec((1, 1), lambda i: (0, 0)),
        out_shape=jax.ShapeDtypeStruct((1, 1), jnp.float32),
        scratch_shapes=[pltpu.VMEM((8, H), jnp.float32)],
    )(acc2, p2, degT, Wr1, br1r, Wr2, br2r, Wo, bor)


def kernel(x, edge_index, graph_id, W1, b1, W2, b2, Wr1, br1, Wr2, br2, Wo, bo):
    del graph_id
    src = edge_index[0]
    dst = edge_index[1]

    dst_deg = dst.reshape(2, 16, DEG_ROWS_PER_SUB, DEG_CHUNK)
    src2d = src.reshape(E // AGG_CHUNK, AGG_CHUNK)
    src_idx2 = jnp.stack([src2d, src2d + N])
    dst2d = dst.reshape(E // AGG_CHUNK, AGG_CHUNK)
    zeros_b = jnp.zeros((NPAD, HH), jnp.float32)

    deg2 = _sc_deg(dst_deg)
    degT = deg2[:, :N].T

    b1r = jnp.broadcast_to(b1.reshape(2, 1, HH), (2, 8, HH))
    b2r = jnp.broadcast_to(b2.reshape(2, 1, HH), (2, 8, HH))

    p1 = _tc_mm1(x, W1, b1r, degT)
    acc1 = _sc_agg(p1.reshape(2 * N, HH), src_idx2, dst2d, zeros_b)
    p2 = _tc_mm2(acc1, p1, degT, W2, b2r)
    acc2 = _sc_agg(p2.reshape(2 * N, HH), src_idx2, dst2d, zeros_b)
    return _tc_final(acc2, p2, degT, Wr1, br1.reshape(1, H), Wr2,
                     br2.reshape(1, H), Wo, bo.reshape(1, 1))


# speedup vs baseline: 22.0121x; 1.0005x over previous
import functools

import jax
import jax.numpy as jnp
from jax import lax
from jax.experimental import pallas as pl
from jax.experimental.pallas import tpu as pltpu
from jax.experimental.pallas import tpu_sc as plsc

N = 10000
E = 320000
D = 128
H = 256
HH = 128

NPAD = 10240

DEG_CHUNK = 80
DEG_ROWS_PER_SUB = 125

AGG_CHUNK = 125
AGG_ROWS_PER_SUB = 160
ROWS_PER_TILE = 640

_MESH = plsc.VectorSubcoreMesh(core_axis_name="c", subcore_axis_name="s")


@functools.partial(
    pl.kernel,
    out_type=jax.ShapeDtypeStruct((2, NPAD), jnp.float32),
    mesh=_MESH,
    scratch_types=[
        pltpu.VMEM((DEG_ROWS_PER_SUB, DEG_CHUNK), jnp.int32),
        pltpu.VMEM((DEG_CHUNK,), jnp.float32),
        pltpu.VMEM((640,), jnp.float32),
        pltpu.VMEM_SHARED((NPAD,), jnp.float32),
    ],
)
def _sc_deg(dst_hbm, out_hbm, idx_v, ones_v, zero_v, deg_sh):
    c = lax.axis_index("c")
    s = lax.axis_index("s")

    pltpu.sync_copy(dst_hbm.at[c, s], idx_v)

    def _fill_ones(k, _):
        ones_v[pl.ds(pl.multiple_of(k * 16, 16), 16)] = jnp.ones((16,), jnp.float32)
        return 0
    lax.fori_loop(0, DEG_CHUNK // 16, _fill_ones, 0)

    def _fill_zero(k, _):
        zero_v[pl.ds(pl.multiple_of(k * 16, 16), 16)] = jnp.zeros((16,), jnp.float32)
        return 0
    lax.fori_loop(0, 40, _fill_zero, 0)

    pltpu.sync_copy(zero_v, deg_sh.at[pl.ds(pl.multiple_of(s * 640, 8), 640)])
    plsc.subcore_barrier()

    def _scat(j, _):
        pltpu.sync_copy(ones_v, deg_sh.at[idx_v.at[j]], add=True)
        return 0
    lax.fori_loop(0, DEG_ROWS_PER_SUB, _scat, 0)

    plsc.subcore_barrier()

    @pl.when(s == 0)
    def _():
        pltpu.sync_copy(deg_sh, out_hbm.at[c])


@functools.partial(
    pl.kernel,
    out_type=jax.ShapeDtypeStruct((2, NPAD, HH), jnp.float32),
    mesh=_MESH,
    scratch_types=[
        pltpu.VMEM((16, AGG_CHUNK), jnp.int32),
        pltpu.VMEM((16, AGG_CHUNK), jnp.int32),
        pltpu.VMEM((AGG_CHUNK, HH), jnp.float32),
        pltpu.VMEM((AGG_CHUNK, HH), jnp.float32),
        pltpu.VMEM_SHARED((NPAD, HH), jnp.float32),
        pltpu.SemaphoreType.DMA,
        pltpu.SemaphoreType.DMA,
        pltpu.SemaphoreType.DMA,
        pltpu.SemaphoreType.DMA,
    ],
)
def _sc_agg(p_hbm, srcidx_hbm, dstidx_hbm, zeros_hbm, out_hbm,
            src_v, dst_v, rows0_v, rows1_v, acc_sh, sem0, sem1, ssem0, ssem1):
    c = lax.axis_index("c")
    s = lax.axis_index("s")

    base = s * AGG_ROWS_PER_SUB

    r0 = pl.multiple_of(s * ROWS_PER_TILE, 8)
    pltpu.sync_copy(zeros_hbm.at[pl.ds(r0, ROWS_PER_TILE)],
                    acc_sh.at[pl.ds(r0, ROWS_PER_TILE)])
    plsc.subcore_barrier()

    rows = (rows0_v, rows1_v)
    sems = (sem0, sem1)
    ssems = (ssem0, ssem1)

    def _group(g, _):
        off = pl.multiple_of(base + g * 16, 8)
        pltpu.sync_copy(srcidx_hbm.at[c, pl.ds(off, 16)], src_v)
        pltpu.sync_copy(dstidx_hbm.at[pl.ds(off, 16)], dst_v)

        gd = [None, None]
        sd = [None, None]
        gd[0] = pltpu.async_copy(p_hbm.at[src_v.at[0]], rows[0], sems[0])
        for j in range(16):
            b = j & 1
            if j < 15:
                if sd[1 - b] is not None:
                    sd[1 - b].wait()
                gd[1 - b] = pltpu.async_copy(
                    p_hbm.at[src_v.at[j + 1]], rows[1 - b], sems[1 - b])
            gd[b].wait()
            sd[b] = pltpu.async_copy(rows[b], acc_sh.at[dst_v.at[j]],
                                     ssems[b], add=True)
        sd[0].wait()
        sd[1].wait()
        return 0
    lax.fori_loop(0, AGG_ROWS_PER_SUB // 16, _group, 0)

    plsc.subcore_barrier()
    pltpu.sync_copy(acc_sh.at[pl.ds(r0, ROWS_PER_TILE)],
                    out_hbm.at[c, pl.ds(r0, ROWS_PER_TILE)])


_BLK = 1000


def _mm1_body(x_ref, w_ref, b_ref, deg_ref, p_ref):
    dinv = lax.rsqrt(deg_ref[:, 0:1] + deg_ref[:, 1:2] + 1.0)
    h = jnp.dot(x_ref[...], w_ref[...], preferred_element_type=jnp.float32)
    p_ref[0] = (h + b_ref[0, 0:1, :]) * dinv


def _tc_mm1(x, W1, b1r, degT):
    return pl.pallas_call(
        _mm1_body,
        grid=(2, N // _BLK),
        in_specs=[
            pl.BlockSpec((_BLK, D), lambda c, i: (i, 0)),
            pl.BlockSpec((D, HH), lambda c, i: (0, c)),
            pl.BlockSpec((1, 8, HH), lambda c, i: (c, 0, 0)),
            pl.BlockSpec((_BLK, 2), lambda c, i: (i, 0)),
        ],
        out_specs=pl.BlockSpec((1, _BLK, HH), lambda c, i: (c, i, 0)),
        out_shape=jax.ShapeDtypeStruct((2, N, HH), jnp.float32),
    )(x, W1, b1r, degT)


def _mm2_body(acc_ref, p_ref, deg_ref, w_ref, b_ref, out_ref):
    dinv = lax.rsqrt(deg_ref[:, 0:1] + deg_ref[:, 1:2] + 1.0)
    h0 = jnp.maximum(dinv * (acc_ref[0] + p_ref[0]), 0.0)
    h1 = jnp.maximum(dinv * (acc_ref[1] + p_ref[1]), 0.0)
    h = jnp.concatenate([h0, h1], axis=1)
    out_ref[0] = (jnp.dot(h, w_ref[...], preferred_element_type=jnp.float32)
                  + b_ref[0, 0:1, :]) * dinv


def _tc_mm2(acc1, p1, degT, W2, b2r):
    return pl.pallas_call(
        _mm2_body,
        grid=(2, N // _BLK),
        in_specs=[
            pl.BlockSpec((2, _BLK, HH), lambda c, i: (0, i, 0)),
            pl.BlockSpec((2, _BLK, HH), lambda c, i: (0, i, 0)),
            pl.BlockSpec((_BLK, 2), lambda c, i: (i, 0)),
            pl.BlockSpec((H, HH), lambda c, i: (0, c)),
            pl.BlockSpec((1, 8, HH), lambda c, i: (c, 0, 0)),
        ],
        out_specs=pl.BlockSpec((1, _BLK, HH), lambda c, i: (c, i, 0)),
        out_shape=jax.ShapeDtypeStruct((2, N, HH), jnp.float32),
    )(acc1, p1, degT, W2, b2r)


def _final_body(acc_ref, p_ref, deg_ref, wr1_ref, br1_ref, wr2_ref, br2_ref,
                wo_ref, bo_ref, out_ref, sum_ref):
    i = pl.program_id(0)
    dinv = lax.rsqrt(deg_ref[:, 0:1] + deg_ref[:, 1:2] + 1.0)
    h0 = jnp.maximum(dinv * (acc_ref[0] + p_ref[0]), 0.0)
    h1 = jnp.maximum(dinv * (acc_ref[1] + p_ref[1]), 0.0)
    part = jnp.concatenate(
        [jnp.sum(h0, axis=0, keepdims=True), jnp.sum(h1, axis=0, keepdims=True)],
        axis=1)

    @pl.when(i == 0)
    def _():
        sum_ref[0:1] = part

    @pl.when(i > 0)
    def _():
        sum_ref[0:1] = sum_ref[0:1] + part

    @pl.when(i == pl.num_programs(0) - 1)
    def _():
        pooled = sum_ref[0:1] * (1.0 / N)
        r = jnp.dot(pooled, wr1_ref[...],
                    preferred_element_type=jnp.float32) + br1_ref[...]
        r = jnp.dot(r, wr2_ref[...],
                    preferred_element_type=jnp.float32) + br2_ref[...]
        out_ref[...] = jnp.dot(r, wo_ref[...],
                               preferred_element_type=jnp.float32) + bo_ref[...]


def _tc_final(acc2, p2, degT, Wr1, br1r, Wr2, br2r, Wo, bor):
    return pl.pallas_call(
        _final_body,
        grid=(N // _BLK,),
        in_specs=[
            pl.BlockSpec((2, _BLK, HH), lambda i: (0, i, 0)),
            pl.BlockSpec((2, _BLK, HH), lambda i: (0, i, 0)),
            pl.BlockSpec((_BLK, 2), lambda i: (i, 0)),
            pl.BlockSpec((H, H), lambda i: (0, 0)),
            pl.BlockSpec((1, H), lambda i: (0, 0)),
            pl.BlockSpec((H, H), lambda i: (0, 0)),
            pl.BlockSpec((1, H), lambda i: (0, 0)),
            pl.BlockSpec((H, 1), lambda i: (0, 0)),
            pl.BlockSpec((1, 1), lambda i: (0, 0)),
        ],
        out_specs=pl.BlockSpec((1, 1), lambda i: (0, 0)),
        out_shape=jax.ShapeDtypeStruct((1, 1), jnp.float32),
        scratch_shapes=[pltpu.VMEM((8, H), jnp.float32)],
    )(acc2, p2, degT, Wr1, br1r, Wr2, br2r, Wo, bor)


def kernel(x, edge_index, graph_id, W1, b1, W2, b2, Wr1, br1, Wr2, br2, Wo, bo):
    del graph_id
    src = edge_index[0]
    dst = edge_index[1]

    dst_deg = dst.reshape(2, 16, DEG_ROWS_PER_SUB, DEG_CHUNK)
    src2d = src.reshape(E // AGG_CHUNK, AGG_CHUNK)
    src_idx2 = jnp.stack([src2d, src2d + N])
    dst2d = dst.reshape(E // AGG_CHUNK, AGG_CHUNK)
    zeros_b = jnp.zeros((NPAD, HH), jnp.float32)

    deg2 = _sc_deg(dst_deg)
    degT = deg2[:, :N].T

    b1r = jnp.broadcast_to(b1.reshape(2, 1, HH), (2, 8, HH))
    b2r = jnp.broadcast_to(b2.reshape(2, 1, HH), (2, 8, HH))

    p1 = _tc_mm1(x, W1, b1r, degT)
    acc1 = _sc_agg(p1.reshape(2 * N, HH), src_idx2, dst2d, zeros_b)
    p2 = _tc_mm2(acc1, p1, degT, W2, b2r)
    acc2 = _sc_agg(p2.reshape(2 * N, HH), src_idx2, dst2d, zeros_b)
    return _tc_final(acc2, p2, degT, Wr1, br1.reshape(1, H), Wr2,
                     br2.reshape(1, H), Wo, bo.reshape(1, 1))
